# baseline (speedup 1.0000x reference)
import functools

import jax
import jax.numpy as jnp
from jax import lax
from jax.experimental import pallas as pl
from jax.experimental.pallas import tpu as pltpu
from jax.experimental.pallas import tpu_sc as plsc

_NC = 2
_NS = 16
_L = 16
_NW = _NC * _NS
_B = 16384
_D = 64
_BPW = _B // _NW
_CHUNK = 128
_NCHUNK = _BPW // _CHUNK


def _sqrt16(x):
    xs = jnp.maximum(x, jnp.float32(1e-30))
    i = plsc.bitcast(xs, jnp.int32)
    i = jnp.int32(0x5F3759DF) - (i >> 1)
    y = plsc.bitcast(i, jnp.float32)
    half = xs * jnp.float32(0.5)
    for _ in range(3):
        y = y * (jnp.float32(1.5) - half * y * y)
    return xs * y


def _transe_body(head_hbm, rel_hbm, tail_hbm, ent_hbm, relemb_hbm, out_hbm,
                 hidx, ridx, tidx, hrows, rrows, trows, tscr, oloc, sem):
    wid = lax.axis_index("s") * _NC + lax.axis_index("c")
    base = wid * _BPW

    pltpu.sync_copy(head_hbm.at[wid], hidx)
    pltpu.sync_copy(rel_hbm.at[wid], ridx)
    pltpu.sync_copy(tail_hbm.at[wid], tidx)

    copies = []
    for c in range(_NCHUNK):
        sl = pl.ds(c * _CHUNK, _CHUNK)
        copies.append(pltpu.async_copy(ent_hbm.at[hidx.at[c]], hrows.at[sl], sem))
        copies.append(pltpu.async_copy(relemb_hbm.at[ridx.at[c]], rrows.at[sl], sem))
        copies.append(pltpu.async_copy(ent_hbm.at[tidx.at[c]], trows.at[sl], sem))
    for cp in copies:
        cp.wait()

    lanes = lax.iota(jnp.int32, _L)

    def block_body(b, carry):
        rbase = b * _L
        for r in range(_L):
            row = rbase + r
            acc = None
            for cc in range(_D // _L):
                ds = pl.ds(cc * _L, _L)
                d = hrows[row, ds] + rrows[row, ds] - trows[row, ds]
                sq = d * d
                acc = sq if acc is None else acc + sq
            tscr[pl.ds(r * _L, _L)] = acc
        s = None
        for col in range(_L):
            g = plsc.load_gather(tscr, [lanes * _L + col])
            s = g if s is None else s + g
        oloc[pl.ds(rbase, _L)] = _sqrt16(s)
        return carry

    lax.fori_loop(0, _BPW // _L, block_body, 0)
    pltpu.sync_copy(oloc, out_hbm.at[pl.ds(base, _BPW)])


_transe = functools.partial(
    pl.kernel,
    out_type=jax.ShapeDtypeStruct((_B,), jnp.float32),
    mesh=plsc.VectorSubcoreMesh(core_axis_name="c", subcore_axis_name="s",
                                num_cores=_NC, num_subcores=_NS),
    compiler_params=pltpu.CompilerParams(needs_layout_passes=False,
                                         use_tc_tiling_on_sc=False),
    scratch_types=[
        pltpu.VMEM((_NCHUNK, _CHUNK), jnp.int32),
        pltpu.VMEM((_NCHUNK, _CHUNK), jnp.int32),
        pltpu.VMEM((_NCHUNK, _CHUNK), jnp.int32),
        pltpu.VMEM((_BPW, _D), jnp.float32),
        pltpu.VMEM((_BPW, _D), jnp.float32),
        pltpu.VMEM((_BPW, _D), jnp.float32),
        pltpu.VMEM((_L * _L,), jnp.float32),
        pltpu.VMEM((_BPW,), jnp.float32),
        pltpu.SemaphoreType.DMA,
    ],
)(_transe_body)


def kernel(head, rel, tail, ent_emb, rel_emb):
    h = head.astype(jnp.int32).reshape(_NW, _NCHUNK, _CHUNK)
    r = rel.astype(jnp.int32).reshape(_NW, _NCHUNK, _CHUNK)
    t = tail.astype(jnp.int32).reshape(_NW, _NCHUNK, _CHUNK)
    return _transe(h, r, t, ent_emb, rel_emb)
